# baseline (speedup 1.0000x reference)
import functools

import jax
import jax.numpy as jnp
from jax import lax
from jax.experimental import pallas as pl
from jax.experimental.pallas import tpu as pltpu
from jax.experimental.pallas import tpu_sc as plsc

N = 10000
NP = 10240
E = 320000
D = 128
G = 64
NC = 2
NS = 16
NW = NC * NS
CHUNK = 128
NCHUNKS = E // CHUNK
ITERS = -(-NCHUNKS // NW)
R = 640
GRID = NP // R
GA = 72

_mesh = plsc.VectorSubcoreMesh(
    core_axis_name="c", subcore_axis_name="s", num_cores=NC, num_subcores=NS)


MAXC = -(-NCHUNKS // NW) + 1
HALF = 40
NCPAD = NCHUNKS + 8


@functools.partial(
    pl.kernel,
    out_type=jax.ShapeDtypeStruct((NC, NP, D), jnp.float32),
    mesh=_mesh,
    compiler_params=pltpu.CompilerParams(needs_layout_passes=False),
    scratch_types=[
        pltpu.VMEM_SHARED((NP, D), jnp.float32),
        pltpu.VMEM((HALF, 1, CHUNK), jnp.int32),
        pltpu.VMEM((HALF, 1, CHUNK), jnp.int32),
        pltpu.VMEM((CHUNK, D), jnp.float32),
        pltpu.VMEM((CHUNK, D), jnp.float32),
        pltpu.SemaphoreType.DMA,
        pltpu.SemaphoreType.DMA,
        pltpu.SemaphoreType.DMA,
        pltpu.SemaphoreType.DMA,
    ],
)
def _sc_aggregate(h_hbm, src_hbm, dst_hbm, out_hbm, acc, srcb, dstb,
                  rows_a, rows_b, sem_a, sem_b, ssem_a, ssem_b):
    c = lax.axis_index("c")
    s = lax.axis_index("s")
    w = c * NS + s
    lo = (w * NCHUNKS) // NW
    cnt = ((w + 1) * NCHUNKS) // NW - lo

    def _zrow(r, _):
        for j in range(D // 16):
            rows_a[r, pl.ds(j * 16, 16)] = jnp.zeros((16,), jnp.float32)
        return 0
    lax.fori_loop(0, CHUNK, _zrow, 0)
    for k in range(NP // NS // CHUNK):
        pltpu.sync_copy(rows_a, acc.at[pl.ds((s * 5 + k) * CHUNK, CHUNK)])
    plsc.subcore_barrier()

    def _wait_g(q, buf, sem):
        pltpu.make_async_copy(h_hbm.at[srcb.at[q, 0]], buf, sem).wait()

    def _wait_s(q, buf, sem):
        pltpu.make_async_copy(buf, acc.at[dstb.at[q, 0]], sem).wait()

    for half in range(2):
        base = half * HALF

        @pl.when(cnt > base)
        def _():
            pltpu.sync_copy(src_hbm.at[pl.ds(lo + base, HALF)], srcb)
            pltpu.sync_copy(dst_hbm.at[pl.ds(lo + base, HALF)], dstb)
            pltpu.async_copy(h_hbm.at[srcb.at[0, 0]], rows_a, sem_a)

        @pl.when(cnt > base + 1)
        def _():
            pltpu.async_copy(h_hbm.at[srcb.at[1, 0]], rows_b, sem_b)

        def _pair(q2, _):
            q = 2 * q2
            k = base + q

            @pl.when(k < cnt)
            def _():
                _wait_g(q, rows_a, sem_a)
                pltpu.async_copy(rows_a, acc.at[dstb.at[q, 0]], ssem_a,
                                 add=True)

            @pl.when(k + 1 < cnt)
            def _():
                _wait_g(q + 1, rows_b, sem_b)
                pltpu.async_copy(rows_b, acc.at[dstb.at[q + 1, 0]], ssem_b,
                                 add=True)

            @pl.when((k + 2 < cnt) & (q + 2 < HALF))
            def _():
                _wait_s(q, rows_a, ssem_a)
                pltpu.async_copy(h_hbm.at[srcb.at[q + 2, 0]], rows_a, sem_a)

            @pl.when((k + 3 < cnt) & (q + 3 < HALF))
            def _():
                _wait_s(q + 1, rows_b, ssem_b)
                pltpu.async_copy(h_hbm.at[srcb.at[q + 3, 0]], rows_b, sem_b)
            return 0
        lax.fori_loop(0, HALF // 2, _pair, 0)

        @pl.when(cnt > base)
        def _():
            _wait_s(0, rows_a, ssem_a)

        @pl.when(cnt > base + 1)
        def _():
            _wait_s(1, rows_b, ssem_b)
    plsc.subcore_barrier()

    pltpu.sync_copy(acc.at[pl.ds(s * (NP // NS), NP // NS)],
                    out_hbm.at[c, pl.ds(s * (NP // NS), NP // NS)])


@functools.partial(
    pl.kernel,
    out_type=jax.ShapeDtypeStruct((NC, G, D), jnp.float32),
    mesh=_mesh,
    compiler_params=pltpu.CompilerParams(needs_layout_passes=False),
    scratch_types=[
        pltpu.VMEM_SHARED((NS, GA, D), jnp.float32),
        pltpu.VMEM((1, GA, D), jnp.float32),
        pltpu.VMEM((64, D), jnp.float32),
        pltpu.VMEM((64,), jnp.int32),
        pltpu.VMEM((NS, 1, D), jnp.float32),
        pltpu.VMEM((1, D), jnp.float32),
    ],
)
def _sc_maxpool(h_hbm, batch_hbm, out_hbm, parts, acc, hrows, bvals,
                comb_v, res_v):
    c = lax.axis_index("c")
    s = lax.axis_index("s")
    w = c * NS + s
    zero16 = jnp.zeros((16,), jnp.int32)
    lanes = lax.iota(jnp.int32, 16)

    def _zrow(r, _):
        for j in range(D // 16):
            acc[0, r, pl.ds(j * 16, 16)] = jnp.zeros((16,), jnp.float32)
        return 0
    lax.fori_loop(0, GA, _zrow, 0)

    rows_per_tile = NP // NW
    for ch in range(rows_per_tile // 64):
        row0 = w * rows_per_tile + ch * 64
        pltpu.sync_copy(h_hbm.at[pl.ds(row0, 64)], hrows)
        pltpu.sync_copy(batch_hbm.at[pl.ds(row0, 64)], bvals)

        def _row(r, _):
            bvec = plsc.load_gather(bvals, [jnp.full((16,), r, jnp.int32)])
            for j in range(D // 16):
                col = lanes + j * 16
                cur = plsc.load_gather(acc, [zero16, bvec, col])
                val = hrows[r, pl.ds(j * 16, 16)]
                plsc.store_scatter(acc, [zero16, bvec, col],
                                   jnp.maximum(cur, val))
            return 0
        lax.fori_loop(0, 64, _row, 0)

    pltpu.sync_copy(acc, parts.at[pl.ds(s, 1)])
    plsc.subcore_barrier()

    for k in range(G // NS):
        g = s * (G // NS) + k
        pltpu.sync_copy(parts.at[:, pl.ds(g, 1), :], comb_v)

        def _red(t, m):
            return tuple(
                jnp.maximum(m[j], comb_v[t, 0, pl.ds(j * 16, 16)])
                for j in range(D // 16))
        m = lax.fori_loop(0, NS, _red,
                          tuple(jnp.zeros((16,), jnp.float32)
                                for _ in range(D // 16)))
        for j in range(D // 16):
            res_v[0, pl.ds(j * 16, 16)] = m[j]
        pltpu.sync_copy(res_v, out_hbm.at[c, pl.ds(g, 1)])


def _dense_body(p_ref, x_ref, b3_ref, wrelT_ref, brel_ref, wrootT_ref,
                h_ref, sum_ref, cnt_ref):
    i = pl.program_id(0)
    agg = p_ref[0] + p_ref[1]
    h = jnp.dot(agg, wrelT_ref[...], preferred_element_type=jnp.float32)
    h += jnp.dot(x_ref[...], wrootT_ref[...],
                 preferred_element_type=jnp.float32)
    h += brel_ref[...]
    h = jnp.maximum(h, 0.0)
    rows = lax.broadcasted_iota(jnp.int32, (R, 1), 0) + i * R
    h = jnp.where(rows < N, h, 0.0)
    h_ref[...] = h

    b = b3_ref[0]
    gids = lax.broadcasted_iota(jnp.int32, (G, R), 0)
    P = (b == gids).astype(jnp.float32)
    blk_sum = lax.dot_general(P, h, (((1,), (0,)), ((), ())),
                              preferred_element_type=jnp.float32)
    blk_cnt = lax.dot_general(P, jnp.ones((R, D), jnp.float32),
                              (((1,), (0,)), ((), ())),
                              preferred_element_type=jnp.float32)

    @pl.when(i == 0)
    def _():
        sum_ref[...] = jnp.zeros_like(sum_ref)
        cnt_ref[...] = jnp.zeros_like(cnt_ref)
    sum_ref[...] += blk_sum
    cnt_ref[...] += blk_cnt


def _tc_dense(partials, h_in, batch3, wrelT, brel, wrootT):
    return pl.pallas_call(
        _dense_body,
        grid=(GRID,),
        in_specs=[
            pl.BlockSpec((NC, R, D), lambda i: (0, i, 0)),
            pl.BlockSpec((R, D), lambda i: (i, 0)),
            pl.BlockSpec((1, 1, R), lambda i: (i, 0, 0)),
            pl.BlockSpec((D, D), lambda i: (0, 0)),
            pl.BlockSpec((1, D), lambda i: (0, 0)),
            pl.BlockSpec((D, D), lambda i: (0, 0)),
        ],
        out_specs=[
            pl.BlockSpec((R, D), lambda i: (i, 0)),
            pl.BlockSpec((G, D), lambda i: (0, 0)),
            pl.BlockSpec((G, D), lambda i: (0, 0)),
        ],
        out_shape=[
            jax.ShapeDtypeStruct((NP, D), jnp.float32),
            jax.ShapeDtypeStruct((G, D), jnp.float32),
            jax.ShapeDtypeStruct((G, D), jnp.float32),
        ],
    )(partials, h_in, batch3, wrelT, brel, wrootT)


def _head_body(m1_ref, m2_ref, m3_ref, s1_ref, s2_ref, s3_ref, cnt_ref,
               w1T_ref, b1_ref, w2T_ref, b2_ref, w3T_ref, b3_ref,
               out_ref, g_ref):
    cnt = cnt_ref[...]
    ssum = s1_ref[...] + s2_ref[...] + s3_ref[...]
    mean = ssum / jnp.maximum(cnt, 1.0)
    mx = (jnp.maximum(m1_ref[0], m1_ref[1])
          + jnp.maximum(m2_ref[0], m2_ref[1])
          + jnp.maximum(m3_ref[0], m3_ref[1]))
    mx = jnp.where(cnt > 0, mx, 0.0)
    g = jnp.concatenate([mx, mean], axis=1)
    a = jnp.maximum(jnp.dot(g, w1T_ref[...],
                            preferred_element_type=jnp.float32)
                    + b1_ref[...], 0.0)
    a = jnp.maximum(jnp.dot(a, w2T_ref[...],
                            preferred_element_type=jnp.float32)
                    + b2_ref[...], 0.0)
    out_ref[...] = (jnp.dot(a, w3T_ref[...],
                            preferred_element_type=jnp.float32)
                    + b3_ref[...])
    g_ref[...] = g


def _tc_head(m1, m2, m3, s1, s2, s3, cnt, w1T, b1, w2T, b2, w3T, b3):
    return pl.pallas_call(
        _head_body,
        out_shape=[
            jax.ShapeDtypeStruct((G, 10), jnp.float32),
            jax.ShapeDtypeStruct((G, 2 * D), jnp.float32),
        ],
    )(m1, m2, m3, s1, s2, s3, cnt, w1T, b1, w2T, b2, w3T, b3)


def kernel(x, edge_index, batch, Wrel1, brel1, Wroot1, Wrel2, brel2, Wroot2,
           Wrel3, brel3, Wroot3, W1, b1, W2, b2, W3, b3):
    x_p = jnp.pad(x, ((0, NP - N), (0, 0)))
    src = jnp.pad(edge_index[0].reshape(NCHUNKS, 1, CHUNK),
                  ((0, NCPAD - NCHUNKS), (0, 0), (0, 0)))
    dst = jnp.pad(edge_index[1].reshape(NCHUNKS, 1, CHUNK),
                  ((0, NCPAD - NCHUNKS), (0, 0), (0, 0)))
    batch_p = jnp.pad(batch, (0, NP - N), constant_values=G)
    batch3 = batch_p.reshape(GRID, 1, R)

    p1 = _sc_aggregate(x_p, src, dst)
    h1, s1, cnt = _tc_dense(p1, x_p, batch3, Wrel1.T, brel1[None], Wroot1.T)
    m1 = _sc_maxpool(h1, batch_p)

    p2 = _sc_aggregate(h1, src, dst)
    h2, s2, _ = _tc_dense(p2, h1, batch3, Wrel2.T, brel2[None], Wroot2.T)
    m2 = _sc_maxpool(h2, batch_p)

    p3 = _sc_aggregate(h2, src, dst)
    h3, s3, _ = _tc_dense(p3, h2, batch3, Wrel3.T, brel3[None], Wroot3.T)
    m3 = _sc_maxpool(h3, batch_p)

    out, g = _tc_head(m1, m2, m3, s1, s2, s3, cnt,
                      W1.T, b1[None], W2.T, b2[None], W3.T, b3[None])
    node_embs = lax.stop_gradient(h3[:N])
    return (out, (node_embs, lax.stop_gradient(g)))


# speedup vs baseline: 1.0839x; 1.0839x over previous
import functools

import jax
import jax.numpy as jnp
from jax import lax
from jax.experimental import pallas as pl
from jax.experimental.pallas import tpu as pltpu
from jax.experimental.pallas import tpu_sc as plsc

N = 10000
NP = 10240
E = 320000
D = 128
G = 64
NC = 2
NS = 16
NW = NC * NS
CHUNK = 128
NCHUNKS = E // CHUNK
ITERS = -(-NCHUNKS // NW)
R = 640
GRID = NP // R
GA = 72

_mesh = plsc.VectorSubcoreMesh(
    core_axis_name="c", subcore_axis_name="s", num_cores=NC, num_subcores=NS)


MAXC = -(-NCHUNKS // NW) + 1
HALF = 40
NCPAD = NCHUNKS + 8


@functools.partial(
    pl.kernel,
    out_type=jax.ShapeDtypeStruct((NC, NP, D), jnp.float32),
    mesh=_mesh,
    compiler_params=pltpu.CompilerParams(needs_layout_passes=False),
    scratch_types=[
        pltpu.VMEM_SHARED((NP, D), jnp.float32),
        pltpu.VMEM((HALF, 1, CHUNK), jnp.int32),
        pltpu.VMEM((HALF, 1, CHUNK), jnp.int32),
        pltpu.VMEM((CHUNK, D), jnp.float32),
        pltpu.VMEM((CHUNK, D), jnp.float32),
        pltpu.SemaphoreType.DMA,
        pltpu.SemaphoreType.DMA,
    ],
)
def _sc_aggregate(h_hbm, src_hbm, dst_hbm, out_hbm, acc, srcb, dstb,
                  rows_a, rows_b, sem_a, sem_b):
    c = lax.axis_index("c")
    s = lax.axis_index("s")
    w = c * NS + s
    lo = (w * NCHUNKS) // NW
    cnt = ((w + 1) * NCHUNKS) // NW - lo

    def _zrow(r, _):
        for j in range(D // 16):
            rows_a[r, pl.ds(j * 16, 16)] = jnp.zeros((16,), jnp.float32)
        return 0
    lax.fori_loop(0, CHUNK, _zrow, 0)
    for k in range(NP // NS // CHUNK):
        pltpu.sync_copy(rows_a, acc.at[pl.ds((s * 5 + k) * CHUNK, CHUNK)])
    plsc.subcore_barrier()

    def _wait_g(q, buf, sem):
        pltpu.make_async_copy(h_hbm.at[srcb.at[q, 0]], buf, sem).wait()

    for half in range(2):
        base = half * HALF

        @pl.when(cnt > base)
        def _():
            pltpu.sync_copy(src_hbm.at[pl.ds(lo + base, HALF)], srcb)
            pltpu.sync_copy(dst_hbm.at[pl.ds(lo + base, HALF)], dstb)
            pltpu.async_copy(h_hbm.at[srcb.at[0, 0]], rows_a, sem_a)

        def _pair(q2, _):
            q = 2 * q2
            k = base + q

            @pl.when(k < cnt)
            def _():
                _wait_g(q, rows_a, sem_a)

                @pl.when((k + 1 < cnt) & (q + 1 < HALF))
                def _():
                    pltpu.async_copy(h_hbm.at[srcb.at[q + 1, 0]], rows_b,
                                     sem_b)
                pltpu.sync_copy(rows_a, acc.at[dstb.at[q, 0]], add=True)

            @pl.when(k + 1 < cnt)
            def _():
                _wait_g(q + 1, rows_b, sem_b)

                @pl.when((k + 2 < cnt) & (q + 2 < HALF))
                def _():
                    pltpu.async_copy(h_hbm.at[srcb.at[q + 2, 0]], rows_a,
                                     sem_a)
                pltpu.sync_copy(rows_b, acc.at[dstb.at[q + 1, 0]], add=True)
            return 0
        lax.fori_loop(0, HALF // 2, _pair, 0)
    plsc.subcore_barrier()

    pltpu.sync_copy(acc.at[pl.ds(s * (NP // NS), NP // NS)],
                    out_hbm.at[c, pl.ds(s * (NP // NS), NP // NS)])


@functools.partial(
    pl.kernel,
    out_type=jax.ShapeDtypeStruct((NC, G, D), jnp.float32),
    mesh=_mesh,
    compiler_params=pltpu.CompilerParams(needs_layout_passes=False),
    scratch_types=[
        pltpu.VMEM_SHARED((NS, GA, D), jnp.float32),
        pltpu.VMEM((1, GA, D), jnp.float32),
        pltpu.VMEM((64, D), jnp.float32),
        pltpu.VMEM((64,), jnp.int32),
        pltpu.VMEM((NS, 1, D), jnp.float32),
        pltpu.VMEM((1, D), jnp.float32),
    ],
)
def _sc_maxpool(h_hbm, batch_hbm, out_hbm, parts, acc, hrows, bvals,
                comb_v, res_v):
    c = lax.axis_index("c")
    s = lax.axis_index("s")
    w = c * NS + s
    zero16 = jnp.zeros((16,), jnp.int32)
    lanes = lax.iota(jnp.int32, 16)

    def _zrow(r, _):
        for j in range(D // 16):
            acc[0, r, pl.ds(j * 16, 16)] = jnp.zeros((16,), jnp.float32)
        return 0
    lax.fori_loop(0, GA, _zrow, 0)

    rows_per_tile = NP // NW
    for ch in range(rows_per_tile // 64):
        row0 = w * rows_per_tile + ch * 64
        pltpu.sync_copy(h_hbm.at[pl.ds(row0, 64)], hrows)
        pltpu.sync_copy(batch_hbm.at[pl.ds(row0, 64)], bvals)

        def _row(r, _):
            bvec = plsc.load_gather(bvals, [jnp.full((16,), r, jnp.int32)])
            for j in range(D // 16):
                col = lanes + j * 16
                cur = plsc.load_gather(acc, [zero16, bvec, col])
                val = hrows[r, pl.ds(j * 16, 16)]
                plsc.store_scatter(acc, [zero16, bvec, col],
                                   jnp.maximum(cur, val))
            return 0
        lax.fori_loop(0, 64, _row, 0)

    pltpu.sync_copy(acc, parts.at[pl.ds(s, 1)])
    plsc.subcore_barrier()

    for k in range(G // NS):
        g = s * (G // NS) + k
        pltpu.sync_copy(parts.at[:, pl.ds(g, 1), :], comb_v)

        def _red(t, m):
            return tuple(
                jnp.maximum(m[j], comb_v[t, 0, pl.ds(j * 16, 16)])
                for j in range(D // 16))
        m = lax.fori_loop(0, NS, _red,
                          tuple(jnp.zeros((16,), jnp.float32)
                                for _ in range(D // 16)))
        for j in range(D // 16):
            res_v[0, pl.ds(j * 16, 16)] = m[j]
        pltpu.sync_copy(res_v, out_hbm.at[c, pl.ds(g, 1)])


def _dense_body(p_ref, x_ref, b3_ref, wrelT_ref, brel_ref, wrootT_ref,
                h_ref, sum_ref, cnt_ref):
    i = pl.program_id(0)
    agg = p_ref[0] + p_ref[1]
    h = jnp.dot(agg, wrelT_ref[...], preferred_element_type=jnp.float32)
    h += jnp.dot(x_ref[...], wrootT_ref[...],
                 preferred_element_type=jnp.float32)
    h += brel_ref[...]
    h = jnp.maximum(h, 0.0)
    rows = lax.broadcasted_iota(jnp.int32, (R, 1), 0) + i * R
    h = jnp.where(rows < N, h, 0.0)
    h_ref[...] = h

    b = b3_ref[0]
    gids = lax.broadcasted_iota(jnp.int32, (G, R), 0)
    P = (b == gids).astype(jnp.float32)
    blk_sum = lax.dot_general(P, h, (((1,), (0,)), ((), ())),
                              preferred_element_type=jnp.float32)
    blk_cnt = lax.dot_general(P, jnp.ones((R, D), jnp.float32),
                              (((1,), (0,)), ((), ())),
                              preferred_element_type=jnp.float32)

    @pl.when(i == 0)
    def _():
        sum_ref[...] = jnp.zeros_like(sum_ref)
        cnt_ref[...] = jnp.zeros_like(cnt_ref)
    sum_ref[...] += blk_sum
    cnt_ref[...] += blk_cnt


def _tc_dense(partials, h_in, batch3, wrelT, brel, wrootT):
    return pl.pallas_call(
        _dense_body,
        grid=(GRID,),
        in_specs=[
            pl.BlockSpec((NC, R, D), lambda i: (0, i, 0)),
            pl.BlockSpec((R, D), lambda i: (i, 0)),
            pl.BlockSpec((1, 1, R), lambda i: (i, 0, 0)),
            pl.BlockSpec((D, D), lambda i: (0, 0)),
            pl.BlockSpec((1, D), lambda i: (0, 0)),
            pl.BlockSpec((D, D), lambda i: (0, 0)),
        ],
        out_specs=[
            pl.BlockSpec((R, D), lambda i: (i, 0)),
            pl.BlockSpec((G, D), lambda i: (0, 0)),
            pl.BlockSpec((G, D), lambda i: (0, 0)),
        ],
        out_shape=[
            jax.ShapeDtypeStruct((NP, D), jnp.float32),
            jax.ShapeDtypeStruct((G, D), jnp.float32),
            jax.ShapeDtypeStruct((G, D), jnp.float32),
        ],
    )(partials, h_in, batch3, wrelT, brel, wrootT)


def _head_body(m1_ref, m2_ref, m3_ref, s1_ref, s2_ref, s3_ref, cnt_ref,
               w1T_ref, b1_ref, w2T_ref, b2_ref, w3T_ref, b3_ref,
               out_ref, g_ref):
    cnt = cnt_ref[...]
    ssum = s1_ref[...] + s2_ref[...] + s3_ref[...]
    mean = ssum / jnp.maximum(cnt, 1.0)
    mx = (jnp.maximum(m1_ref[0], m1_ref[1])
          + jnp.maximum(m2_ref[0], m2_ref[1])
          + jnp.maximum(m3_ref[0], m3_ref[1]))
    mx = jnp.where(cnt > 0, mx, 0.0)
    g = jnp.concatenate([mx, mean], axis=1)
    a = jnp.maximum(jnp.dot(g, w1T_ref[...],
                            preferred_element_type=jnp.float32)
                    + b1_ref[...], 0.0)
    a = jnp.maximum(jnp.dot(a, w2T_ref[...],
                            preferred_element_type=jnp.float32)
                    + b2_ref[...], 0.0)
    out_ref[...] = (jnp.dot(a, w3T_ref[...],
                            preferred_element_type=jnp.float32)
                    + b3_ref[...])
    g_ref[...] = g


def _tc_head(m1, m2, m3, s1, s2, s3, cnt, w1T, b1, w2T, b2, w3T, b3):
    return pl.pallas_call(
        _head_body,
        out_shape=[
            jax.ShapeDtypeStruct((G, 10), jnp.float32),
            jax.ShapeDtypeStruct((G, 2 * D), jnp.float32),
        ],
    )(m1, m2, m3, s1, s2, s3, cnt, w1T, b1, w2T, b2, w3T, b3)


def kernel(x, edge_index, batch, Wrel1, brel1, Wroot1, Wrel2, brel2, Wroot2,
           Wrel3, brel3, Wroot3, W1, b1, W2, b2, W3, b3):
    x_p = jnp.pad(x, ((0, NP - N), (0, 0)))
    src = jnp.pad(edge_index[0].reshape(NCHUNKS, 1, CHUNK),
                  ((0, NCPAD - NCHUNKS), (0, 0), (0, 0)))
    dst = jnp.pad(edge_index[1].reshape(NCHUNKS, 1, CHUNK),
                  ((0, NCPAD - NCHUNKS), (0, 0), (0, 0)))
    batch_p = jnp.pad(batch, (0, NP - N), constant_values=G)
    batch3 = batch_p.reshape(GRID, 1, R)

    p1 = _sc_aggregate(x_p, src, dst)
    h1, s1, cnt = _tc_dense(p1, x_p, batch3, Wrel1.T, brel1[None], Wroot1.T)
    m1 = _sc_maxpool(h1, batch_p)

    p2 = _sc_aggregate(h1, src, dst)
    h2, s2, _ = _tc_dense(p2, h1, batch3, Wrel2.T, brel2[None], Wroot2.T)
    m2 = _sc_maxpool(h2, batch_p)

    p3 = _sc_aggregate(h2, src, dst)
    h3, s3, _ = _tc_dense(p3, h2, batch3, Wrel3.T, brel3[None], Wroot3.T)
    m3 = _sc_maxpool(h3, batch_p)

    out, g = _tc_head(m1, m2, m3, s1, s2, s3, cnt,
                      W1.T, b1[None], W2.T, b2[None], W3.T, b3[None])
    node_embs = lax.stop_gradient(h3[:N])
    return (out, (node_embs, lax.stop_gradient(g)))


# speedup vs baseline: 1.0986x; 1.0136x over previous
import functools

import jax
import jax.numpy as jnp
from jax import lax
from jax.experimental import pallas as pl
from jax.experimental.pallas import tpu as pltpu
from jax.experimental.pallas import tpu_sc as plsc

N = 10000
NP = 10240
E = 320000
D = 128
G = 64
NC = 2
NS = 16
NW = NC * NS
CHUNK = 128
NCHUNKS = E // CHUNK
ITERS = -(-NCHUNKS // NW)
R = 640
GRID = NP // R
GA = 72

_mesh = plsc.VectorSubcoreMesh(
    core_axis_name="c", subcore_axis_name="s", num_cores=NC, num_subcores=NS)


MAXC = -(-NCHUNKS // NW) + 1
HALF = 40
NCPAD = NCHUNKS + 8


STAGE = 16
NSTAGE = 5
MRPP = 8


@functools.partial(
    pl.kernel,
    out_type=(jax.ShapeDtypeStruct((NC, NP, D), jnp.float32),
              jax.ShapeDtypeStruct((NC, NS, GA, D), jnp.float32)),
    mesh=_mesh,
    compiler_params=pltpu.CompilerParams(needs_layout_passes=False),
    scratch_types=[
        pltpu.VMEM_SHARED((NP, D), jnp.float32),
        pltpu.VMEM((STAGE, 1, CHUNK), jnp.int32),
        pltpu.VMEM((STAGE, 1, CHUNK), jnp.int32),
        pltpu.VMEM((CHUNK, D), jnp.float32),
        pltpu.VMEM((CHUNK, D), jnp.float32),
        pltpu.VMEM((1, GA, D), jnp.float32),
        pltpu.VMEM((MRPP, D), jnp.float32),
        pltpu.VMEM((MRPP, D), jnp.float32),
        pltpu.VMEM((NP // NW,), jnp.int32),
        pltpu.SemaphoreType.DMA,
        pltpu.SemaphoreType.DMA,
        pltpu.SemaphoreType.DMA,
        pltpu.SemaphoreType.DMA,
    ],
)
def _sc_agg_pool(h_hbm, src_hbm, dst_hbm, batch_hbm, out_hbm, mx_hbm,
                 acc, srcb, dstb, rows_a, rows_b, macc, hr_a, hr_b, bvals,
                 sem_a, sem_b, sem_ha, sem_hb):
    c = lax.axis_index("c")
    s = lax.axis_index("s")
    w = c * NS + s
    lo = (w * NCHUNKS) // NW
    cnt = ((w + 1) * NCHUNKS) // NW - lo
    rpt = NP // NW
    r0 = w * rpt
    lanes = lax.iota(jnp.int32, 16)
    zero16 = jnp.zeros((16,), jnp.int32)
    npairs = NSTAGE * (STAGE // 2)

    def _zrow(r, _):
        for j in range(D // 16):
            rows_a[r, pl.ds(j * 16, 16)] = jnp.zeros((16,), jnp.float32)
        return 0
    lax.fori_loop(0, CHUNK, _zrow, 0)
    for k in range(NP // NS // CHUNK):
        pltpu.sync_copy(rows_a, acc.at[pl.ds((s * 5 + k) * CHUNK, CHUNK)])

    def _zmac(r, _):
        for j in range(D // 16):
            macc[0, r, pl.ds(j * 16, 16)] = jnp.zeros((16,), jnp.float32)
        return 0
    lax.fori_loop(0, GA, _zmac, 0)
    pltpu.sync_copy(batch_hbm.at[pl.ds(r0, rpt)], bvals)
    pltpu.async_copy(h_hbm.at[pl.ds(r0, MRPP)], hr_a, sem_ha)
    plsc.subcore_barrier()

    def _wait_g(q, buf, sem):
        pltpu.make_async_copy(h_hbm.at[srcb.at[q, 0]], buf, sem).wait()

    def _pool_rows(pg, hr, sem, nxt, sem_nxt):
        pltpu.make_async_copy(h_hbm.at[pl.ds(r0 + pg * MRPP, MRPP)], hr,
                              sem).wait()

        @pl.when(pg + 1 < npairs)
        def _():
            pltpu.async_copy(h_hbm.at[pl.ds(r0 + (pg + 1) * MRPP, MRPP)],
                             nxt, sem_nxt)
        for i in range(MRPP):
            bvec = plsc.load_gather(bvals, [jnp.full((16,), pg * MRPP + i,
                                                     jnp.int32)])
            for j in range(D // 16):
                col = lanes + j * 16
                cur = plsc.load_gather(macc, [zero16, bvec, col])
                plsc.store_scatter(macc, [zero16, bvec, col],
                                   jnp.maximum(cur, hr[i, pl.ds(j * 16, 16)]))

    for stage in range(NSTAGE):
        base = stage * STAGE

        @pl.when(cnt > base)
        def _():
            pltpu.sync_copy(src_hbm.at[pl.ds(lo + base, STAGE)], srcb)
            pltpu.sync_copy(dst_hbm.at[pl.ds(lo + base, STAGE)], dstb)
            pltpu.async_copy(h_hbm.at[srcb.at[0, 0]], rows_a, sem_a)

        def _quad(u, _):
            for half_pair in range(2):
                q = 4 * u + 2 * half_pair
                k = base + q
                pg = stage * (STAGE // 2) + 2 * u + half_pair
                hr, shr = (hr_a, sem_ha) if half_pair == 0 else (hr_b, sem_hb)
                nx, snx = (hr_b, sem_hb) if half_pair == 0 else (hr_a, sem_ha)

                @pl.when(k < cnt)
                def _():
                    _wait_g(q, rows_a, sem_a)

                    @pl.when((k + 1 < cnt) & (q + 1 < STAGE))
                    def _():
                        pltpu.async_copy(h_hbm.at[srcb.at[q + 1, 0]], rows_b,
                                         sem_b)
                    pltpu.sync_copy(rows_a, acc.at[dstb.at[q, 0]], add=True)
                _pool_rows(pg, hr, shr, nx, snx)

                @pl.when(k + 1 < cnt)
                def _():
                    _wait_g(q + 1, rows_b, sem_b)

                    @pl.when((k + 2 < cnt) & (q + 2 < STAGE))
                    def _():
                        pltpu.async_copy(h_hbm.at[srcb.at[q + 2, 0]], rows_a,
                                         sem_a)
                    pltpu.sync_copy(rows_b, acc.at[dstb.at[q + 1, 0]],
                                    add=True)
            return 0
        lax.fori_loop(0, STAGE // 4, _quad, 0)
    plsc.subcore_barrier()

    pltpu.sync_copy(acc.at[pl.ds(s * (NP // NS), NP // NS)],
                    out_hbm.at[c, pl.ds(s * (NP // NS), NP // NS)])
    pltpu.sync_copy(macc, mx_hbm.at[c, pl.ds(s, 1)])


@functools.partial(
    pl.kernel,
    out_type=jax.ShapeDtypeStruct((NC, G, D), jnp.float32),
    mesh=_mesh,
    compiler_params=pltpu.CompilerParams(needs_layout_passes=False),
    scratch_types=[
        pltpu.VMEM_SHARED((NS, GA, D), jnp.float32),
        pltpu.VMEM((1, GA, D), jnp.float32),
        pltpu.VMEM((64, D), jnp.float32),
        pltpu.VMEM((64,), jnp.int32),
        pltpu.VMEM((NS, 1, D), jnp.float32),
        pltpu.VMEM((1, D), jnp.float32),
    ],
)
def _sc_maxpool(h_hbm, batch_hbm, out_hbm, parts, acc, hrows, bvals,
                comb_v, res_v):
    c = lax.axis_index("c")
    s = lax.axis_index("s")
    w = c * NS + s
    zero16 = jnp.zeros((16,), jnp.int32)
    lanes = lax.iota(jnp.int32, 16)

    def _zrow(r, _):
        for j in range(D // 16):
            acc[0, r, pl.ds(j * 16, 16)] = jnp.zeros((16,), jnp.float32)
        return 0
    lax.fori_loop(0, GA, _zrow, 0)

    rows_per_tile = NP // NW
    for ch in range(rows_per_tile // 64):
        row0 = w * rows_per_tile + ch * 64
        pltpu.sync_copy(h_hbm.at[pl.ds(row0, 64)], hrows)
        pltpu.sync_copy(batch_hbm.at[pl.ds(row0, 64)], bvals)

        def _row(r, _):
            bvec = plsc.load_gather(bvals, [jnp.full((16,), r, jnp.int32)])
            for j in range(D // 16):
                col = lanes + j * 16
                cur = plsc.load_gather(acc, [zero16, bvec, col])
                val = hrows[r, pl.ds(j * 16, 16)]
                plsc.store_scatter(acc, [zero16, bvec, col],
                                   jnp.maximum(cur, val))
            return 0
        lax.fori_loop(0, 64, _row, 0)

    pltpu.sync_copy(acc, parts.at[pl.ds(s, 1)])
    plsc.subcore_barrier()

    for k in range(G // NS):
        g = s * (G // NS) + k
        pltpu.sync_copy(parts.at[:, pl.ds(g, 1), :], comb_v)

        def _red(t, m):
            return tuple(
                jnp.maximum(m[j], comb_v[t, 0, pl.ds(j * 16, 16)])
                for j in range(D // 16))
        m = lax.fori_loop(0, NS, _red,
                          tuple(jnp.zeros((16,), jnp.float32)
                                for _ in range(D // 16)))
        for j in range(D // 16):
            res_v[0, pl.ds(j * 16, 16)] = m[j]
        pltpu.sync_copy(res_v, out_hbm.at[c, pl.ds(g, 1)])


def _dense_body(p_ref, x_ref, b3_ref, wrelT_ref, brel_ref, wrootT_ref,
                h_ref, sum_ref, cnt_ref):
    i = pl.program_id(0)
    agg = p_ref[0] + p_ref[1]
    h = jnp.dot(agg, wrelT_ref[...], preferred_element_type=jnp.float32)
    h += jnp.dot(x_ref[...], wrootT_ref[...],
                 preferred_element_type=jnp.float32)
    h += brel_ref[...]
    h = jnp.maximum(h, 0.0)
    rows = lax.broadcasted_iota(jnp.int32, (R, 1), 0) + i * R
    h = jnp.where(rows < N, h, 0.0)
    h_ref[...] = h

    b = b3_ref[0]
    gids = lax.broadcasted_iota(jnp.int32, (G, R), 0)
    P = (b == gids).astype(jnp.float32)
    blk_sum = lax.dot_general(P, h, (((1,), (0,)), ((), ())),
                              preferred_element_type=jnp.float32)
    blk_cnt = lax.dot_general(P, jnp.ones((R, D), jnp.float32),
                              (((1,), (0,)), ((), ())),
                              preferred_element_type=jnp.float32)

    @pl.when(i == 0)
    def _():
        sum_ref[...] = jnp.zeros_like(sum_ref)
        cnt_ref[...] = jnp.zeros_like(cnt_ref)
    sum_ref[...] += blk_sum
    cnt_ref[...] += blk_cnt


def _tc_dense(partials, h_in, batch3, wrelT, brel, wrootT):
    return pl.pallas_call(
        _dense_body,
        grid=(GRID,),
        in_specs=[
            pl.BlockSpec((NC, R, D), lambda i: (0, i, 0)),
            pl.BlockSpec((R, D), lambda i: (i, 0)),
            pl.BlockSpec((1, 1, R), lambda i: (i, 0, 0)),
            pl.BlockSpec((D, D), lambda i: (0, 0)),
            pl.BlockSpec((1, D), lambda i: (0, 0)),
            pl.BlockSpec((D, D), lambda i: (0, 0)),
        ],
        out_specs=[
            pl.BlockSpec((R, D), lambda i: (i, 0)),
            pl.BlockSpec((G, D), lambda i: (0, 0)),
            pl.BlockSpec((G, D), lambda i: (0, 0)),
        ],
        out_shape=[
            jax.ShapeDtypeStruct((NP, D), jnp.float32),
            jax.ShapeDtypeStruct((G, D), jnp.float32),
            jax.ShapeDtypeStruct((G, D), jnp.float32),
        ],
    )(partials, h_in, batch3, wrelT, brel, wrootT)


def _head_body(m1_ref, m2_ref, m3_ref, s1_ref, s2_ref, s3_ref, cnt_ref,
               w1T_ref, b1_ref, w2T_ref, b2_ref, w3T_ref, b3_ref,
               out_ref, g_ref):
    cnt = cnt_ref[...]
    ssum = s1_ref[...] + s2_ref[...] + s3_ref[...]
    mean = ssum / jnp.maximum(cnt, 1.0)

    def _redmax(ref):
        m = ref[0]
        for t in range(1, NW):
            m = jnp.maximum(m, ref[t])
        return m[:G]

    mx = (_redmax(m1_ref) + _redmax(m2_ref)
          + jnp.maximum(m3_ref[0], m3_ref[1]))
    mx = jnp.where(cnt > 0, mx, 0.0)
    g = jnp.concatenate([mx, mean], axis=1)
    a = jnp.maximum(jnp.dot(g, w1T_ref[...],
                            preferred_element_type=jnp.float32)
                    + b1_ref[...], 0.0)
    a = jnp.maximum(jnp.dot(a, w2T_ref[...],
                            preferred_element_type=jnp.float32)
                    + b2_ref[...], 0.0)
    out_ref[...] = (jnp.dot(a, w3T_ref[...],
                            preferred_element_type=jnp.float32)
                    + b3_ref[...])
    g_ref[...] = g


def _tc_head(m1, m2, m3, s1, s2, s3, cnt, w1T, b1, w2T, b2, w3T, b3):
    return pl.pallas_call(
        _head_body,
        out_shape=[
            jax.ShapeDtypeStruct((G, 10), jnp.float32),
            jax.ShapeDtypeStruct((G, 2 * D), jnp.float32),
        ],
    )(m1, m2, m3, s1, s2, s3, cnt, w1T, b1, w2T, b2, w3T, b3)


def kernel(x, edge_index, batch, Wrel1, brel1, Wroot1, Wrel2, brel2, Wroot2,
           Wrel3, brel3, Wroot3, W1, b1, W2, b2, W3, b3):
    x_p = jnp.pad(x, ((0, NP - N), (0, 0)))
    src = jnp.pad(edge_index[0].reshape(NCHUNKS, 1, CHUNK),
                  ((0, NCPAD - NCHUNKS), (0, 0), (0, 0)))
    dst = jnp.pad(edge_index[1].reshape(NCHUNKS, 1, CHUNK),
                  ((0, NCPAD - NCHUNKS), (0, 0), (0, 0)))
    batch_p = jnp.pad(batch, (0, NP - N), constant_values=G)
    batch3 = batch_p.reshape(GRID, 1, R)

    p1, _ = _sc_agg_pool(x_p, src, dst, batch_p)
    h1, s1, cnt = _tc_dense(p1, x_p, batch3, Wrel1.T, brel1[None], Wroot1.T)

    p2, mp1 = _sc_agg_pool(h1, src, dst, batch_p)
    h2, s2, _ = _tc_dense(p2, h1, batch3, Wrel2.T, brel2[None], Wroot2.T)

    p3, mp2 = _sc_agg_pool(h2, src, dst, batch_p)
    h3, s3, _ = _tc_dense(p3, h2, batch3, Wrel3.T, brel3[None], Wroot3.T)
    m3 = _sc_maxpool(h3, batch_p)

    out, g = _tc_head(mp1.reshape(NW, GA, D), mp2.reshape(NW, GA, D), m3,
                      s1, s2, s3, cnt,
                      W1.T, b1[None], W2.T, b2[None], W3.T, b3[None])
    node_embs = lax.stop_gradient(h3[:N])
    return (out, (node_embs, lax.stop_gradient(g)))


# speedup vs baseline: 1.1056x; 1.0063x over previous
import functools

import jax
import jax.numpy as jnp
from jax import lax
from jax.experimental import pallas as pl
from jax.experimental.pallas import tpu as pltpu
from jax.experimental.pallas import tpu_sc as plsc

N = 10000
NP = 10240
E = 320000
D = 128
G = 64
NC = 2
NS = 16
NW = NC * NS
CHUNK = 128
NCHUNKS = E // CHUNK
ITERS = -(-NCHUNKS // NW)
R = 640
GRID = NP // R
GA = 72

_mesh = plsc.VectorSubcoreMesh(
    core_axis_name="c", subcore_axis_name="s", num_cores=NC, num_subcores=NS)


MAXC = -(-NCHUNKS // NW) + 1
HALF = 40
NCPAD = NCHUNKS + 8


STAGE = 16
NSTAGE = 5
MRPP = 8


@functools.partial(
    pl.kernel,
    out_type=(jax.ShapeDtypeStruct((NC, NP, D), jnp.float32),
              jax.ShapeDtypeStruct((NC, NS, GA, D), jnp.float32)),
    mesh=_mesh,
    compiler_params=pltpu.CompilerParams(needs_layout_passes=False),
    scratch_types=[
        pltpu.VMEM_SHARED((NP, D), jnp.float32),
        pltpu.VMEM((STAGE, 1, CHUNK), jnp.int32),
        pltpu.VMEM((STAGE, 1, CHUNK), jnp.int32),
        pltpu.VMEM((CHUNK, D), jnp.float32),
        pltpu.VMEM((CHUNK, D), jnp.float32),
        pltpu.VMEM((1, GA, D), jnp.float32),
        pltpu.VMEM((MRPP, D), jnp.float32),
        pltpu.VMEM((MRPP, D), jnp.float32),
        pltpu.VMEM((NP // NW,), jnp.int32),
        pltpu.SemaphoreType.DMA,
        pltpu.SemaphoreType.DMA,
        pltpu.SemaphoreType.DMA,
        pltpu.SemaphoreType.DMA,
        pltpu.SemaphoreType.DMA,
    ],
)
def _sc_agg_pool(h_hbm, src_hbm, dst_hbm, batch_hbm, out_hbm, mx_hbm,
                 acc, srcb, dstb, rows_a, rows_b, macc, hr_a, hr_b, bvals,
                 sem_a, sem_b, sem_ha, sem_hb, ssem):
    c = lax.axis_index("c")
    s = lax.axis_index("s")
    w = c * NS + s
    lo = (w * NCHUNKS) // NW
    cnt = ((w + 1) * NCHUNKS) // NW - lo
    rpt = NP // NW
    r0 = w * rpt
    lanes = lax.iota(jnp.int32, 16)
    zero16 = jnp.zeros((16,), jnp.int32)
    npairs = NSTAGE * (STAGE // 2)

    def _zrow(r, _):
        for j in range(D // 16):
            rows_a[r, pl.ds(j * 16, 16)] = jnp.zeros((16,), jnp.float32)
        return 0
    lax.fori_loop(0, CHUNK, _zrow, 0)
    for k in range(NP // NS // CHUNK):
        pltpu.sync_copy(rows_a, acc.at[pl.ds((s * 5 + k) * CHUNK, CHUNK)])

    def _zmac(r, _):
        for j in range(D // 16):
            macc[0, r, pl.ds(j * 16, 16)] = jnp.zeros((16,), jnp.float32)
        return 0
    lax.fori_loop(0, GA, _zmac, 0)
    pltpu.sync_copy(batch_hbm.at[pl.ds(r0, rpt)], bvals)
    pltpu.async_copy(h_hbm.at[pl.ds(r0, MRPP)], hr_a, sem_ha)
    plsc.subcore_barrier()

    def _wait_g(q, buf, sem):
        pltpu.make_async_copy(h_hbm.at[srcb.at[q, 0]], buf, sem).wait()

    def _pool_rows(pg, hr, sem, nxt, sem_nxt, part):
        if part == 0:
            pltpu.make_async_copy(h_hbm.at[pl.ds(r0 + pg * MRPP, MRPP)], hr,
                                  sem).wait()

            @pl.when(pg + 1 < npairs)
            def _():
                pltpu.async_copy(h_hbm.at[pl.ds(r0 + (pg + 1) * MRPP, MRPP)],
                                 nxt, sem_nxt)
        for i in range(part * (MRPP // 2), (part + 1) * (MRPP // 2)):
            bvec = plsc.load_gather(bvals, [jnp.full((16,), pg * MRPP + i,
                                                     jnp.int32)])
            for j in range(D // 16):
                col = lanes + j * 16
                cur = plsc.load_gather(macc, [zero16, bvec, col])
                plsc.store_scatter(macc, [zero16, bvec, col],
                                   jnp.maximum(cur, hr[i, pl.ds(j * 16, 16)]))

    for stage in range(NSTAGE):
        base = stage * STAGE

        @pl.when(cnt > base)
        def _():
            pltpu.sync_copy(src_hbm.at[pl.ds(lo + base, STAGE)], srcb)
            pltpu.sync_copy(dst_hbm.at[pl.ds(lo + base, STAGE)], dstb)
            pltpu.async_copy(h_hbm.at[srcb.at[0, 0]], rows_a, sem_a)

        def _quad(u, _):
            for half_pair in range(2):
                q = 4 * u + 2 * half_pair
                k = base + q
                pg = stage * (STAGE // 2) + 2 * u + half_pair
                hr, shr = (hr_a, sem_ha) if half_pair == 0 else (hr_b, sem_hb)
                nx, snx = (hr_b, sem_hb) if half_pair == 0 else (hr_a, sem_ha)

                @pl.when(k < cnt)
                def _():
                    _wait_g(q, rows_a, sem_a)

                    @pl.when((k + 1 < cnt) & (q + 1 < STAGE))
                    def _():
                        pltpu.async_copy(h_hbm.at[srcb.at[q + 1, 0]], rows_b,
                                         sem_b)
                    pltpu.async_copy(rows_a, acc.at[dstb.at[q, 0]], ssem,
                                     add=True)
                _pool_rows(pg, hr, shr, nx, snx, 0)

                @pl.when(k < cnt)
                def _():
                    pltpu.make_async_copy(rows_a, acc.at[dstb.at[q, 0]],
                                          ssem).wait()

                @pl.when(k + 1 < cnt)
                def _():
                    _wait_g(q + 1, rows_b, sem_b)

                    @pl.when((k + 2 < cnt) & (q + 2 < STAGE))
                    def _():
                        pltpu.async_copy(h_hbm.at[srcb.at[q + 2, 0]], rows_a,
                                         sem_a)
                    pltpu.async_copy(rows_b, acc.at[dstb.at[q + 1, 0]], ssem,
                                     add=True)
                _pool_rows(pg, hr, shr, nx, snx, 1)

                @pl.when(k + 1 < cnt)
                def _():
                    pltpu.make_async_copy(rows_b, acc.at[dstb.at[q + 1, 0]],
                                          ssem).wait()
            return 0
        lax.fori_loop(0, STAGE // 4, _quad, 0)
    plsc.subcore_barrier()

    pltpu.sync_copy(acc.at[pl.ds(s * (NP // NS), NP // NS)],
                    out_hbm.at[c, pl.ds(s * (NP // NS), NP // NS)])
    pltpu.sync_copy(macc, mx_hbm.at[c, pl.ds(s, 1)])


@functools.partial(
    pl.kernel,
    out_type=jax.ShapeDtypeStruct((NC, G, D), jnp.float32),
    mesh=_mesh,
    compiler_params=pltpu.CompilerParams(needs_layout_passes=False),
    scratch_types=[
        pltpu.VMEM_SHARED((NS, GA, D), jnp.float32),
        pltpu.VMEM((1, GA, D), jnp.float32),
        pltpu.VMEM((64, D), jnp.float32),
        pltpu.VMEM((64,), jnp.int32),
        pltpu.VMEM((NS, 1, D), jnp.float32),
        pltpu.VMEM((1, D), jnp.float32),
    ],
)
def _sc_maxpool(h_hbm, batch_hbm, out_hbm, parts, acc, hrows, bvals,
                comb_v, res_v):
    c = lax.axis_index("c")
    s = lax.axis_index("s")
    w = c * NS + s
    zero16 = jnp.zeros((16,), jnp.int32)
    lanes = lax.iota(jnp.int32, 16)

    def _zrow(r, _):
        for j in range(D // 16):
            acc[0, r, pl.ds(j * 16, 16)] = jnp.zeros((16,), jnp.float32)
        return 0
    lax.fori_loop(0, GA, _zrow, 0)

    rows_per_tile = NP // NW
    for ch in range(rows_per_tile // 64):
        row0 = w * rows_per_tile + ch * 64
        pltpu.sync_copy(h_hbm.at[pl.ds(row0, 64)], hrows)
        pltpu.sync_copy(batch_hbm.at[pl.ds(row0, 64)], bvals)

        def _row(r, _):
            bvec = plsc.load_gather(bvals, [jnp.full((16,), r, jnp.int32)])
            for j in range(D // 16):
                col = lanes + j * 16
                cur = plsc.load_gather(acc, [zero16, bvec, col])
                val = hrows[r, pl.ds(j * 16, 16)]
                plsc.store_scatter(acc, [zero16, bvec, col],
                                   jnp.maximum(cur, val))
            return 0
        lax.fori_loop(0, 64, _row, 0)

    pltpu.sync_copy(acc, parts.at[pl.ds(s, 1)])
    plsc.subcore_barrier()

    for k in range(G // NS):
        g = s * (G // NS) + k
        pltpu.sync_copy(parts.at[:, pl.ds(g, 1), :], comb_v)

        def _red(t, m):
            return tuple(
                jnp.maximum(m[j], comb_v[t, 0, pl.ds(j * 16, 16)])
                for j in range(D // 16))
        m = lax.fori_loop(0, NS, _red,
                          tuple(jnp.zeros((16,), jnp.float32)
                                for _ in range(D // 16)))
        for j in range(D // 16):
            res_v[0, pl.ds(j * 16, 16)] = m[j]
        pltpu.sync_copy(res_v, out_hbm.at[c, pl.ds(g, 1)])


def _dense_body(p_ref, x_ref, b3_ref, wrelT_ref, brel_ref, wrootT_ref,
                h_ref, sum_ref, cnt_ref):
    i = pl.program_id(0)
    agg = p_ref[0] + p_ref[1]
    h = jnp.dot(agg, wrelT_ref[...], preferred_element_type=jnp.float32)
    h += jnp.dot(x_ref[...], wrootT_ref[...],
                 preferred_element_type=jnp.float32)
    h += brel_ref[...]
    h = jnp.maximum(h, 0.0)
    rows = lax.broadcasted_iota(jnp.int32, (R, 1), 0) + i * R
    h = jnp.where(rows < N, h, 0.0)
    h_ref[...] = h

    b = b3_ref[0]
    gids = lax.broadcasted_iota(jnp.int32, (G, R), 0)
    P = (b == gids).astype(jnp.float32)
    blk_sum = lax.dot_general(P, h, (((1,), (0,)), ((), ())),
                              preferred_element_type=jnp.float32)
    blk_cnt = lax.dot_general(P, jnp.ones((R, D), jnp.float32),
                              (((1,), (0,)), ((), ())),
                              preferred_element_type=jnp.float32)

    @pl.when(i == 0)
    def _():
        sum_ref[...] = jnp.zeros_like(sum_ref)
        cnt_ref[...] = jnp.zeros_like(cnt_ref)
    sum_ref[...] += blk_sum
    cnt_ref[...] += blk_cnt


def _tc_dense(partials, h_in, batch3, wrelT, brel, wrootT):
    return pl.pallas_call(
        _dense_body,
        grid=(GRID,),
        in_specs=[
            pl.BlockSpec((NC, R, D), lambda i: (0, i, 0)),
            pl.BlockSpec((R, D), lambda i: (i, 0)),
            pl.BlockSpec((1, 1, R), lambda i: (i, 0, 0)),
            pl.BlockSpec((D, D), lambda i: (0, 0)),
            pl.BlockSpec((1, D), lambda i: (0, 0)),
            pl.BlockSpec((D, D), lambda i: (0, 0)),
        ],
        out_specs=[
            pl.BlockSpec((R, D), lambda i: (i, 0)),
            pl.BlockSpec((G, D), lambda i: (0, 0)),
            pl.BlockSpec((G, D), lambda i: (0, 0)),
        ],
        out_shape=[
            jax.ShapeDtypeStruct((NP, D), jnp.float32),
            jax.ShapeDtypeStruct((G, D), jnp.float32),
            jax.ShapeDtypeStruct((G, D), jnp.float32),
        ],
    )(partials, h_in, batch3, wrelT, brel, wrootT)


def _head_body(m1_ref, m2_ref, m3_ref, s1_ref, s2_ref, s3_ref, cnt_ref,
               w1T_ref, b1_ref, w2T_ref, b2_ref, w3T_ref, b3_ref,
               out_ref, g_ref):
    cnt = cnt_ref[...]
    ssum = s1_ref[...] + s2_ref[...] + s3_ref[...]
    mean = ssum / jnp.maximum(cnt, 1.0)

    def _redmax(ref):
        m = ref[0]
        for t in range(1, NW):
            m = jnp.maximum(m, ref[t])
        return m[:G]

    mx = (_redmax(m1_ref) + _redmax(m2_ref)
          + jnp.maximum(m3_ref[0], m3_ref[1]))
    mx = jnp.where(cnt > 0, mx, 0.0)
    g = jnp.concatenate([mx, mean], axis=1)
    a = jnp.maximum(jnp.dot(g, w1T_ref[...],
                            preferred_element_type=jnp.float32)
                    + b1_ref[...], 0.0)
    a = jnp.maximum(jnp.dot(a, w2T_ref[...],
                            preferred_element_type=jnp.float32)
                    + b2_ref[...], 0.0)
    out_ref[...] = (jnp.dot(a, w3T_ref[...],
                            preferred_element_type=jnp.float32)
                    + b3_ref[...])
    g_ref[...] = g


def _tc_head(m1, m2, m3, s1, s2, s3, cnt, w1T, b1, w2T, b2, w3T, b3):
    return pl.pallas_call(
        _head_body,
        out_shape=[
            jax.ShapeDtypeStruct((G, 10), jnp.float32),
            jax.ShapeDtypeStruct((G, 2 * D), jnp.float32),
        ],
    )(m1, m2, m3, s1, s2, s3, cnt, w1T, b1, w2T, b2, w3T, b3)


def kernel(x, edge_index, batch, Wrel1, brel1, Wroot1, Wrel2, brel2, Wroot2,
           Wrel3, brel3, Wroot3, W1, b1, W2, b2, W3, b3):
    x_p = jnp.pad(x, ((0, NP - N), (0, 0)))
    src = jnp.pad(edge_index[0].reshape(NCHUNKS, 1, CHUNK),
                  ((0, NCPAD - NCHUNKS), (0, 0), (0, 0)))
    dst = jnp.pad(edge_index[1].reshape(NCHUNKS, 1, CHUNK),
                  ((0, NCPAD - NCHUNKS), (0, 0), (0, 0)))
    batch_p = jnp.pad(batch, (0, NP - N), constant_values=G)
    batch3 = batch_p.reshape(GRID, 1, R)

    p1, _ = _sc_agg_pool(x_p, src, dst, batch_p)
    h1, s1, cnt = _tc_dense(p1, x_p, batch3, Wrel1.T, brel1[None], Wroot1.T)

    p2, mp1 = _sc_agg_pool(h1, src, dst, batch_p)
    h2, s2, _ = _tc_dense(p2, h1, batch3, Wrel2.T, brel2[None], Wroot2.T)

    p3, mp2 = _sc_agg_pool(h2, src, dst, batch_p)
    h3, s3, _ = _tc_dense(p3, h2, batch3, Wrel3.T, brel3[None], Wroot3.T)
    m3 = _sc_maxpool(h3, batch_p)

    out, g = _tc_head(mp1.reshape(NW, GA, D), mp2.reshape(NW, GA, D), m3,
                      s1, s2, s3, cnt,
                      W1.T, b1[None], W2.T, b2[None], W3.T, b3[None])
    node_embs = lax.stop_gradient(h3[:N])
    return (out, (node_embs, lax.stop_gradient(g)))
